# Initial kernel scaffold; baseline (speedup 1.0000x reference)
import jax
import jax.numpy as jnp
from jax.experimental import pallas as pl


def kernel(x, pe):
    raise NotImplementedError("write your pallas kernel here")


# speedup vs baseline: 1.8054x; 1.8054x over previous
import jax
import jax.numpy as jnp
from jax.experimental import pallas as pl


def _add_body(x_ref, pe_ref, o_ref):
    o_ref[...] = x_ref[...] + pe_ref[...][None, :, :]


def kernel(x, pe):
    B, L, D = x.shape
    bs = 512
    grid = (L // bs,)
    return pl.pallas_call(
        _add_body,
        grid=grid,
        in_specs=[
            pl.BlockSpec((B, bs, D), lambda i: (0, i, 0)),
            pl.BlockSpec((bs, D), lambda i: (i, 0)),
        ],
        out_specs=pl.BlockSpec((B, bs, D), lambda i: (0, i, 0)),
        out_shape=jax.ShapeDtypeStruct((B, L, D), x.dtype),
    )(x, pe[:L])
